# baseline (speedup 1.0000x reference)
import functools

import jax
import jax.numpy as jnp
from jax import lax
from jax.experimental import pallas as pl
from jax.experimental.pallas import tpu as pltpu
from jax.experimental.pallas import tpu_sc as plsc

VOCAB_N = 4
LANES = 16
NUM_WORKERS = 32


@functools.cache
def _build(n_flat: int):
    per_worker = n_flat // NUM_WORKERS
    ch = 6400
    assert per_worker % ch == 0
    n_chunks = per_worker // ch

    mesh = plsc.VectorSubcoreMesh(core_axis_name="c", subcore_axis_name="s")

    @functools.partial(
        pl.kernel,
        mesh=mesh,
        out_type=jax.ShapeDtypeStruct((n_flat * VOCAB_N,), jnp.float32),
        scratch_types=[
            pltpu.VMEM((ch,), jnp.float32),
            pltpu.VMEM((ch * VOCAB_N,), jnp.float32),
        ],
        compiler_params=pltpu.CompilerParams(needs_layout_passes=False),
    )
    def onehot(x_hbm, out_hbm, in_v, out_v):
        wid = lax.axis_index("s") * 2 + lax.axis_index("c")
        base = wid * per_worker
        lane_iota = lax.iota(jnp.int32, LANES)

        for c_i in range(n_chunks):
            off = base + c_i * ch
            pltpu.sync_copy(x_hbm.at[pl.ds(off, ch)], in_v)

            def body(k, _):
                v = in_v[pl.ds(k * LANES, LANES)]
                vi = v.astype(jnp.int32)
                idx0 = k * (LANES * VOCAB_N) + VOCAB_N * lane_iota
                for c in range(VOCAB_N):
                    val = jnp.where(vi == c, jnp.float32(1.0), jnp.float32(0.0))
                    plsc.store_scatter(out_v, [idx0 + c], val)
                return 0

            lax.fori_loop(0, ch // LANES, body, 0)
            pltpu.sync_copy(out_v, out_hbm.at[pl.ds(VOCAB_N * off, VOCAB_N * ch)])

    return onehot


def kernel(x):
    rows, cols = x.shape
    n_flat = rows * cols
    out_flat = _build(n_flat)(x.reshape(n_flat))
    return out_flat.reshape(rows, cols, VOCAB_N)


# speedup vs baseline: 1.0304x; 1.0304x over previous
import functools

import jax
import jax.numpy as jnp
from jax import lax
from jax.experimental import pallas as pl
from jax.experimental.pallas import tpu as pltpu
from jax.experimental.pallas import tpu_sc as plsc

VOCAB_N = 4
LANES = 16
NUM_WORKERS = 32


@functools.cache
def _build(n_flat: int):
    per_worker = n_flat // NUM_WORKERS
    ch = 6400
    assert per_worker % ch == 0
    n_chunks = per_worker // ch

    mesh = plsc.VectorSubcoreMesh(core_axis_name="c", subcore_axis_name="s")

    @functools.partial(
        pl.kernel,
        mesh=mesh,
        out_type=jax.ShapeDtypeStruct((n_flat * VOCAB_N,), jnp.float32),
        scratch_types=[
            pltpu.VMEM((ch,), jnp.float32),
            pltpu.VMEM((ch,), jnp.float32),
            pltpu.VMEM((ch * VOCAB_N,), jnp.float32),
            pltpu.VMEM((ch * VOCAB_N,), jnp.float32),
            pltpu.SemaphoreType.DMA((2,)),
            pltpu.SemaphoreType.DMA((2,)),
        ],
        compiler_params=pltpu.CompilerParams(needs_layout_passes=False),
    )
    def onehot(x_hbm, out_hbm, in_v0, in_v1, out_v0, out_v1, in_sem, out_sem):
        wid = lax.axis_index("s") * 2 + lax.axis_index("c")
        base = wid * per_worker
        lane4 = VOCAB_N * lax.iota(jnp.int32, LANES)
        in_bufs = [in_v0, in_v1]
        out_bufs = [out_v0, out_v1]

        def start_in(i):
            off = base + i * ch
            return pltpu.async_copy(
                x_hbm.at[pl.ds(off, ch)], in_bufs[i % 2], in_sem.at[i % 2]
            )

        def start_out(i):
            off = VOCAB_N * (base + i * ch)
            return pltpu.async_copy(
                out_bufs[i % 2], out_hbm.at[pl.ds(off, VOCAB_N * ch)],
                out_sem.at[i % 2],
            )

        in_copies = [start_in(0)]
        out_copies = [None] * n_chunks
        for i in range(n_chunks):
            p = i % 2
            if i + 1 < n_chunks:
                in_copies.append(start_in(i + 1))
            in_copies[i].wait()
            if i >= 2:
                out_copies[i - 2].wait()

            @plsc.parallel_loop(0, ch // LANES, unroll=8)
            def body(k):
                v = in_bufs[p][pl.ds(k * LANES, LANES)]
                idx0 = k * (LANES * VOCAB_N) + lane4
                for c in range(VOCAB_N):
                    val = jnp.where(
                        v == jnp.float32(c), jnp.float32(1.0), jnp.float32(0.0)
                    )
                    plsc.store_scatter(out_bufs[p], [idx0 + c], val)

            out_copies[i] = start_out(i)
        for i in range(max(0, n_chunks - 2), n_chunks):
            out_copies[i].wait()

    return onehot


def kernel(x):
    rows, cols = x.shape
    n_flat = rows * cols
    out_flat = _build(n_flat)(x.reshape(n_flat))
    return out_flat.reshape(rows, cols, VOCAB_N)


# speedup vs baseline: 18.3327x; 17.7925x over previous
import functools

import jax
import jax.numpy as jnp
from jax import lax
from jax.experimental import pallas as pl
from jax.experimental.pallas import tpu as pltpu
from jax.experimental.pallas import tpu_sc as plsc

VOCAB_N = 4
LANES = 16
BLK = 128
NUM_WORKERS = 32


@functools.cache
def _build(n_flat: int):
    per_worker = n_flat // NUM_WORKERS
    ch = 6400
    assert per_worker % ch == 0 and ch % BLK == 0
    n_chunks = per_worker // ch
    blocks = ch // BLK

    mesh = plsc.VectorSubcoreMesh(core_axis_name="c", subcore_axis_name="s")

    @functools.partial(
        pl.kernel,
        mesh=mesh,
        out_type=jax.ShapeDtypeStruct((n_flat * VOCAB_N,), jnp.float32),
        scratch_types=[
            pltpu.VMEM((ch,), jnp.float32),
            pltpu.VMEM((ch,), jnp.float32),
            pltpu.VMEM((ch * VOCAB_N,), jnp.float32),
            pltpu.VMEM((ch * VOCAB_N,), jnp.float32),
            pltpu.SemaphoreType.DMA((2,)),
            pltpu.SemaphoreType.DMA((2,)),
        ],
        compiler_params=pltpu.CompilerParams(needs_layout_passes=False),
    )
    def onehot(x_hbm, out_hbm, in_v0, in_v1, out_v0, out_v1, in_sem, out_sem):
        wid = lax.axis_index("s") * 2 + lax.axis_index("c")
        base = wid * per_worker
        in_bufs = [in_v0, in_v1]
        out_bufs = [out_v0, out_v1]

        def start_in(i):
            off = base + i * ch
            return pltpu.async_copy(
                x_hbm.at[pl.ds(off, ch)], in_bufs[i % 2], in_sem.at[i % 2]
            )

        def start_out(i):
            off = VOCAB_N * (base + i * ch)
            return pltpu.async_copy(
                out_bufs[i % 2], out_hbm.at[pl.ds(off, VOCAB_N * ch)],
                out_sem.at[i % 2],
            )

        in_copies = [start_in(0)]
        out_copies = [None] * n_chunks
        for i in range(n_chunks):
            p = i % 2
            if i + 1 < n_chunks:
                in_copies.append(start_in(i + 1))
            in_copies[i].wait()
            if i >= 2:
                out_copies[i - 2].wait()

            @plsc.parallel_loop(0, blocks, unroll=2)
            def body(b):
                in_off = b * BLK
                out_off = b * (BLK * VOCAB_N)
                for sub in range(BLK // LANES):
                    v = in_bufs[p][pl.ds(in_off + sub * LANES, LANES)]
                    for c in range(VOCAB_N):
                        val = jnp.where(
                            v == jnp.float32(c),
                            jnp.float32(1.0),
                            jnp.float32(0.0),
                        )
                        out_bufs[p][
                            pl.ds(out_off + c * BLK + sub * LANES, LANES)
                        ] = val

            out_copies[i] = start_out(i)
        for i in range(max(0, n_chunks - 2), n_chunks):
            out_copies[i].wait()

    return onehot


def kernel(x):
    rows, cols = x.shape
    n_flat = rows * cols
    xt_flat = x.T.reshape(n_flat)
    out_flat = _build(n_flat)(xt_flat)
    t = out_flat.reshape(cols, rows // BLK, VOCAB_N, BLK)
    return t.transpose(1, 3, 0, 2).reshape(rows, cols, VOCAB_N)
